# Initial kernel scaffold; baseline (speedup 1.0000x reference)
import jax
import jax.numpy as jnp
from jax.experimental import pallas as pl


def kernel(X, orig_X, l_n, idx0, A0, down0, action, W_up, a_s_up, a_n_up, W_end, a_s_end, a_n_end):
    raise NotImplementedError("write your pallas kernel here")


# speedup vs baseline: 2.7088x; 2.7088x over previous
import jax
import jax.numpy as jnp
from jax.experimental import pallas as pl


def _leaky_relu(x, alpha=0.2):
    return jnp.where(x >= 0, x, alpha * x)


def _elu(x):
    return jnp.where(x > 0, x, jnp.exp(jnp.minimum(x, 0.0)) - 1.0)


def _gat_block(feats, a_s_ref, a_n_ref, neg_mask, H, C):
    N = feats.shape[0]
    acc = jnp.zeros((N, C), jnp.float32)
    for h in range(H):
        fh = feats[:, h * C:(h + 1) * C]
        a_s = a_s_ref[h, :]
        a_n = a_n_ref[h, :]
        es = jnp.dot(fh, a_s, preferred_element_type=jnp.float32)
        en = jnp.dot(fh, a_n, preferred_element_type=jnp.float32)
        logits = _leaky_relu(es[:, None] + en[None, :]) + neg_mask
        m = jnp.max(logits, axis=1, keepdims=True)
        p = jnp.exp(logits - m)
        coefs = p / jnp.sum(p, axis=1, keepdims=True)
        acc = acc + jnp.dot(coefs, fh, preferred_element_type=jnp.float32)
    return acc / H


def _body(x_ref, idx_ref, a_ref, down_ref, orig_ref,
          wup_ref, asu_ref, anu_ref, wend_ref, ase_ref, ane_ref, out_ref):
    No, F = x_ref.shape[1], x_ref.shape[2]
    Nn = a_ref.shape[1]
    H, C = asu_ref.shape
    x = x_ref[0]
    idx = idx_ref[0, 0, :]

    rows = jax.lax.broadcasted_iota(jnp.int32, (Nn, No), 0)
    onehot = (rows == idx[None, :]).astype(jnp.float32)
    xu = jnp.dot(onehot, x, preferred_element_type=jnp.float32)

    a = a_ref[0]
    ri = jax.lax.broadcasted_iota(jnp.int32, (Nn, Nn), 0)
    ci = jax.lax.broadcasted_iota(jnp.int32, (Nn, Nn), 1)
    edge = jnp.logical_or(a > 0.5, ri == ci)
    neg_mask = jnp.where(edge, 0.0, -1e9).astype(jnp.float32)

    feats1 = jnp.dot(xu, wup_ref[...], preferred_element_type=jnp.float32)
    x1 = _elu(_gat_block(feats1, asu_ref, anu_ref, neg_mask, H, C))
    x1 = x1 + down_ref[0]

    feats2 = (jnp.dot(x1, wend_ref[:F, :], preferred_element_type=jnp.float32)
              + jnp.dot(orig_ref[0], wend_ref[F:, :],
                        preferred_element_type=jnp.float32))
    out_ref[0] = _elu(_gat_block(feats2, ase_ref, ane_ref, neg_mask, H, C))


def kernel(X, orig_X, l_n, idx0, A0, down0, action, W_up, a_s_up, a_n_up,
           W_end, a_s_end, a_n_end):
    B, No, F = X.shape
    Nn = A0.shape[1]
    H, C = a_s_up.shape
    idx3 = idx0.astype(jnp.int32).reshape(B, 1, No)
    wup = W_up.reshape(F, H * C)
    wend = W_end.reshape(2 * F, H * C)

    full = lambda *shape: pl.BlockSpec(shape, lambda b: (0,) * len(shape))
    out = pl.pallas_call(
        _body,
        grid=(B,),
        in_specs=[
            pl.BlockSpec((1, No, F), lambda b: (b, 0, 0)),
            pl.BlockSpec((1, 1, No), lambda b: (b, 0, 0)),
            pl.BlockSpec((1, Nn, Nn), lambda b: (b, 0, 0)),
            pl.BlockSpec((1, Nn, F), lambda b: (b, 0, 0)),
            pl.BlockSpec((1, Nn, F), lambda b: (b, 0, 0)),
            full(F, H * C),
            full(H, C),
            full(H, C),
            full(2 * F, H * C),
            full(H, C),
            full(H, C),
        ],
        out_specs=pl.BlockSpec((1, Nn, F), lambda b: (b, 0, 0)),
        out_shape=jax.ShapeDtypeStruct((B, Nn, F), jnp.float32),
    )(X, idx3, A0, down0, orig_X, wup, a_s_up, a_n_up, wend, a_s_end, a_n_end)

    scale = (jnp.asarray(l_n) / 1).astype(out.dtype)
    return out * scale


# speedup vs baseline: 3.3058x; 1.2204x over previous
import jax
import jax.numpy as jnp
from jax.experimental import pallas as pl


def _leaky_relu(x, alpha=0.2):
    return jnp.where(x >= 0, x, alpha * x)


def _elu(x):
    return jnp.where(x > 0, x, jnp.exp(jnp.minimum(x, 0.0)) - 1.0)


def _gat_block(feats, a_s_ref, a_n_ref, neg_mask, H, C):
    N = feats.shape[0]
    acc = jnp.zeros((N, C), jnp.float32)
    inv_h = 1.0 / H
    for h in range(H):
        fh = feats[:, h * C:(h + 1) * C]
        a_s = a_s_ref[h, :]
        a_n = a_n_ref[h, :]
        es = jnp.dot(fh, a_s, preferred_element_type=jnp.float32)
        en = jnp.dot(fh, a_n, preferred_element_type=jnp.float32)
        t = es[:, None] + en[None, :]
        p = jnp.exp(jnp.maximum(t, 0.2 * t) + neg_mask)
        s = jnp.sum(p, axis=1, keepdims=True)
        acc = acc + jnp.dot(p, fh,
                            preferred_element_type=jnp.float32) * (inv_h / s)
    return acc


def _body(x_ref, idx_ref, a_ref, down_ref, orig_ref,
          wup_ref, asu_ref, anu_ref, wend_ref, ase_ref, ane_ref, out_ref):
    No, F = x_ref.shape[1], x_ref.shape[2]
    Nn = a_ref.shape[1]
    H, C = asu_ref.shape
    x = x_ref[0]
    idx = idx_ref[0, 0, :]

    rows = jax.lax.broadcasted_iota(jnp.int32, (Nn, No), 0)
    onehot = (rows == idx[None, :]).astype(jnp.float32)
    xu = jnp.dot(onehot, x, preferred_element_type=jnp.float32)

    a = a_ref[0]
    ri = jax.lax.broadcasted_iota(jnp.int32, (Nn, Nn), 0)
    ci = jax.lax.broadcasted_iota(jnp.int32, (Nn, Nn), 1)
    edge = jnp.logical_or(a > 0.5, ri == ci)
    neg_mask = jnp.where(edge, 0.0, -1e9).astype(jnp.float32)

    feats1 = jnp.dot(xu, wup_ref[...], preferred_element_type=jnp.float32)
    x1 = _elu(_gat_block(feats1, asu_ref, anu_ref, neg_mask, H, C))
    x1 = x1 + down_ref[0]

    feats2 = (jnp.dot(x1, wend_ref[:F, :], preferred_element_type=jnp.float32)
              + jnp.dot(orig_ref[0], wend_ref[F:, :],
                        preferred_element_type=jnp.float32))
    out_ref[0] = _elu(_gat_block(feats2, ase_ref, ane_ref, neg_mask, H, C))


def kernel(X, orig_X, l_n, idx0, A0, down0, action, W_up, a_s_up, a_n_up,
           W_end, a_s_end, a_n_end):
    B, No, F = X.shape
    Nn = A0.shape[1]
    H, C = a_s_up.shape
    idx3 = idx0.astype(jnp.int32).reshape(B, 1, No)
    wup = W_up.reshape(F, H * C)
    wend = W_end.reshape(2 * F, H * C)

    full = lambda *shape: pl.BlockSpec(shape, lambda b: (0,) * len(shape))
    out = pl.pallas_call(
        _body,
        grid=(B,),
        in_specs=[
            pl.BlockSpec((1, No, F), lambda b: (b, 0, 0)),
            pl.BlockSpec((1, 1, No), lambda b: (b, 0, 0)),
            pl.BlockSpec((1, Nn, Nn), lambda b: (b, 0, 0)),
            pl.BlockSpec((1, Nn, F), lambda b: (b, 0, 0)),
            pl.BlockSpec((1, Nn, F), lambda b: (b, 0, 0)),
            full(F, H * C),
            full(H, C),
            full(H, C),
            full(2 * F, H * C),
            full(H, C),
            full(H, C),
        ],
        out_specs=pl.BlockSpec((1, Nn, F), lambda b: (b, 0, 0)),
        out_shape=jax.ShapeDtypeStruct((B, Nn, F), jnp.float32),
    )(X, idx3, A0, down0, orig_X, wup, a_s_up, a_n_up, wend, a_s_end, a_n_end)

    scale = (jnp.asarray(l_n) / 1).astype(out.dtype)
    return out * scale


# speedup vs baseline: 3.5715x; 1.0804x over previous
import jax
import jax.numpy as jnp
from jax.experimental import pallas as pl


def _leaky_relu(x, alpha=0.2):
    return jnp.where(x >= 0, x, alpha * x)


def _elu(x):
    return jnp.where(x > 0, x, jnp.exp(jnp.minimum(x, 0.0)) - 1.0)


def _gat_block(feats, a_s_ref, a_n_ref, neg_mask, H, C):
    N = feats.shape[0]
    acc = jnp.zeros((N, C), jnp.float32)
    inv_h = 1.0 / H
    log2e = 1.4426950408889634
    feats_b = feats.astype(jnp.bfloat16)
    for h in range(H):
        fh = feats[:, h * C:(h + 1) * C]
        a_s = a_s_ref[h, :]
        a_n = a_n_ref[h, :]
        es = jnp.dot(fh, a_s, preferred_element_type=jnp.float32) * log2e
        en = jnp.dot(fh, a_n, preferred_element_type=jnp.float32) * log2e
        t = es[:, None] + en[None, :]
        p = jnp.exp2(jnp.maximum(t, 0.2 * t) + neg_mask)
        pb = p.astype(jnp.bfloat16)
        s = jnp.sum(pb, axis=1, keepdims=True,
                    dtype=jnp.float32)
        acc = acc + jnp.dot(pb, feats_b[:, h * C:(h + 1) * C],
                            preferred_element_type=jnp.float32) * (inv_h / s)
    return acc


def _body(x_ref, idx_ref, a_ref, down_ref, orig_ref,
          wup_ref, asu_ref, anu_ref, wend_ref, ase_ref, ane_ref, out_ref):
    No, F = x_ref.shape[1], x_ref.shape[2]
    Nn = a_ref.shape[1]
    H, C = asu_ref.shape
    x = x_ref[0]
    idx = idx_ref[0, 0, :]

    rows = jax.lax.broadcasted_iota(jnp.int32, (Nn, No), 0)
    onehot = (rows == idx[None, :]).astype(jnp.float32)
    xu = jnp.dot(onehot, x, preferred_element_type=jnp.float32)

    a = a_ref[0]
    ri = jax.lax.broadcasted_iota(jnp.int32, (Nn, Nn), 0)
    ci = jax.lax.broadcasted_iota(jnp.int32, (Nn, Nn), 1)
    edge = jnp.logical_or(a > 0.5, ri == ci)
    neg_mask = jnp.where(edge, 0.0, -1e9).astype(jnp.float32)

    feats1 = jnp.dot(xu, wup_ref[...], preferred_element_type=jnp.float32)
    x1 = _elu(_gat_block(feats1, asu_ref, anu_ref, neg_mask, H, C))
    x1 = x1 + down_ref[0]

    feats2 = (jnp.dot(x1, wend_ref[:F, :], preferred_element_type=jnp.float32)
              + jnp.dot(orig_ref[0], wend_ref[F:, :],
                        preferred_element_type=jnp.float32))
    out_ref[0] = _elu(_gat_block(feats2, ase_ref, ane_ref, neg_mask, H, C))


def kernel(X, orig_X, l_n, idx0, A0, down0, action, W_up, a_s_up, a_n_up,
           W_end, a_s_end, a_n_end):
    B, No, F = X.shape
    Nn = A0.shape[1]
    H, C = a_s_up.shape
    idx3 = idx0.astype(jnp.int32).reshape(B, 1, No)
    wup = W_up.reshape(F, H * C)
    wend = W_end.reshape(2 * F, H * C)

    full = lambda *shape: pl.BlockSpec(shape, lambda b: (0,) * len(shape))
    out = pl.pallas_call(
        _body,
        grid=(B,),
        in_specs=[
            pl.BlockSpec((1, No, F), lambda b: (b, 0, 0)),
            pl.BlockSpec((1, 1, No), lambda b: (b, 0, 0)),
            pl.BlockSpec((1, Nn, Nn), lambda b: (b, 0, 0)),
            pl.BlockSpec((1, Nn, F), lambda b: (b, 0, 0)),
            pl.BlockSpec((1, Nn, F), lambda b: (b, 0, 0)),
            full(F, H * C),
            full(H, C),
            full(H, C),
            full(2 * F, H * C),
            full(H, C),
            full(H, C),
        ],
        out_specs=pl.BlockSpec((1, Nn, F), lambda b: (b, 0, 0)),
        out_shape=jax.ShapeDtypeStruct((B, Nn, F), jnp.float32),
    )(X, idx3, A0, down0, orig_X, wup, a_s_up, a_n_up, wend, a_s_end, a_n_end)

    scale = (jnp.asarray(l_n) / 1).astype(out.dtype)
    return out * scale
